# baseline (speedup 1.0000x reference)
import functools

import jax
import jax.numpy as jnp
from jax import lax
from jax.experimental import pallas as pl
from jax.experimental.pallas import tpu as pltpu
from jax.experimental.pallas import tpu_sc as plsc

N_TOKENS = 32768
DIM = 2048
N_EXPERTS = 8
ROUTE_SCALE = 1.0

BLK = 1024

NC = 2
NS = 16
NW = NC * NS
TPW = N_TOKENS // NW
GROUPS = TPW // 16


def _scores_body(x_ref, wt_ref, s_ref):
    s_ref[...] = jnp.dot(x_ref[...], wt_ref[...],
                         preferred_element_type=jnp.float32)


def _tc_scores(x, wt):
    return pl.pallas_call(
        _scores_body,
        grid=(N_TOKENS // BLK,),
        in_specs=[
            pl.BlockSpec((BLK, DIM), lambda i: (i, 0)),
            pl.BlockSpec((DIM, N_EXPERTS), lambda i: (0, 0)),
        ],
        out_specs=pl.BlockSpec((BLK, N_EXPERTS), lambda i: (i, 0)),
        out_shape=jax.ShapeDtypeStruct((N_TOKENS, N_EXPERTS), jnp.float32),
    )(x, wt)


@functools.partial(
    pl.kernel,
    mesh=plsc.VectorSubcoreMesh(core_axis_name="c", subcore_axis_name="s"),
    out_type=[
        jax.ShapeDtypeStruct((N_TOKENS * 2,), jnp.float32),
        jax.ShapeDtypeStruct((N_TOKENS * 2,), jnp.int32),
    ],
    scratch_types=[
        pltpu.VMEM((TPW * N_EXPERTS,), jnp.float32),
        pltpu.VMEM((TPW * 2,), jnp.float32),
        pltpu.VMEM((TPW * 2,), jnp.int32),
    ],
    compiler_params=pltpu.CompilerParams(needs_layout_passes=False),
)
def _sc_router(scores_hbm, w_hbm, i_hbm, s_v, w_v, i_v):
    wid = lax.axis_index("s") * NC + lax.axis_index("c")
    pltpu.sync_copy(scores_hbm.at[pl.ds(wid * (TPW * N_EXPERTS),
                                        TPW * N_EXPERTS)], s_v)

    def group(g, _):
        tok = g * 16 + lax.broadcasted_iota(jnp.int32, (16,), 0)
        s = [plsc.load_gather(s_v, [tok * N_EXPERTS + e])
             for e in range(N_EXPERTS)]
        m1 = s[0]
        i1 = jnp.zeros((16,), jnp.int32)
        m2 = jnp.full((16,), -jnp.inf, jnp.float32)
        i2 = jnp.zeros((16,), jnp.int32)
        for e in range(1, N_EXPERTS):
            se = s[e]
            ev = jnp.full((16,), e, jnp.int32)
            gt1 = se > m1
            gt2 = se > m2
            m2 = jnp.where(gt1, m1, jnp.where(gt2, se, m2))
            i2 = jnp.where(gt1, i1, jnp.where(gt2, ev, i2))
            m1 = jnp.where(gt1, se, m1)
            i1 = jnp.where(gt1, ev, i1)
        denom = jnp.zeros((16,), jnp.float32)
        for e in range(N_EXPERTS):
            denom = denom + jnp.exp(s[e] - m1)
        w1 = (1.0 / denom) * ROUTE_SCALE
        w2 = (jnp.exp(m2 - m1) / denom) * ROUTE_SCALE
        plsc.store_scatter(w_v, [tok * 2], w1)
        plsc.store_scatter(w_v, [tok * 2 + 1], w2)
        plsc.store_scatter(i_v, [tok * 2], i1)
        plsc.store_scatter(i_v, [tok * 2 + 1], i2)
        return 0

    lax.fori_loop(0, GROUPS, group, 0)
    pltpu.sync_copy(w_v, w_hbm.at[pl.ds(wid * (TPW * 2), TPW * 2)])
    pltpu.sync_copy(i_v, i_hbm.at[pl.ds(wid * (TPW * 2), TPW * 2)])


def kernel(x, W):
    scores = _tc_scores(x, W.T)
    w_flat, i_flat = _sc_router(scores.reshape(-1))
    weights = w_flat.reshape(N_TOKENS, 2).astype(x.dtype)
    indices = i_flat.reshape(N_TOKENS, 2)
    return weights, indices
